# baseline (speedup 1.0000x reference)
import functools

import jax
import jax.numpy as jnp
from jax import lax
from jax.experimental import pallas as pl
from jax.experimental.pallas import tpu as pltpu
from jax.experimental.pallas import tpu_sc as plsc

N = 10000
E = 320000
NP = 10240
G = 64
NC = 2
NS = 16
NW = NC * NS
CH = 128
NCHT = 2560
EP = NCHT * CH
RPT = NP // NS

_mesh = functools.partial(
    plsc.VectorSubcoreMesh, core_axis_name="c", subcore_axis_name="s"
)


def _deg_body(dst_hbm, out_hbm, idx2d, deg_v):
    c = lax.axis_index("c")
    s = lax.axis_index("s")
    wid = c * NS + s
    cpt = NCHT // NW

    def zero(i, _):
        deg_v[pl.ds(i * 16, 16)] = jnp.zeros((16,), jnp.float32)
        return 0

    lax.fori_loop(0, NP // 16, zero, 0)

    pltpu.sync_copy(dst_hbm.at[pl.ds(wid * cpt, cpt)], idx2d)
    ones = jnp.ones((16,), jnp.float32)

    def row(r, _):
        def col(k, _):
            idx = idx2d[r, pl.ds(k * 16, 16)]
            plsc.addupdate_scatter(deg_v, [idx], ones)
            return 0

        lax.fori_loop(0, CH // 16, col, 0)
        return 0

    lax.fori_loop(0, cpt, row, 0)
    pltpu.sync_copy(deg_v, out_hbm.at[wid])


def _deg_call(dst2d):
    return pl.kernel(
        _deg_body,
        out_type=jax.ShapeDtypeStruct((NW, NP), jnp.float32),
        mesh=_mesh(),
        scratch_types=[
            pltpu.VMEM((NCHT // NW, CH), jnp.int32),
            pltpu.VMEM((NP,), jnp.float32),
        ],
        compiler_params=pltpu.CompilerParams(needs_layout_passes=False),
    )(dst2d)


def _spmm_body(
    FH, split, xs_hbm, src_hbm, dst_hbm, out_hbm,
    idxs, idxd, rows0, rows1, acc, sem0, sem1,
):
    c = lax.axis_index("c")
    s = lax.axis_index("s")
    if split:
        cpt = NCHT // NS
        base = s * cpt
        tbl = xs_hbm.at[c]
    else:
        cpt = NCHT // NW
        base = (c * NS + s) * cpt
        tbl = xs_hbm

    def zr(r, _):
        def zc(k, _):
            rows0[r, pl.ds(k * 16, 16)] = jnp.zeros((16,), jnp.float32)
            return 0

        lax.fori_loop(0, FH // 16, zc, 0)
        return 0

    lax.fori_loop(0, CH, zr, 0)
    for j in range(RPT // CH):
        pltpu.sync_copy(rows0, acc.at[pl.ds(s * RPT + j * CH, CH)])

    pltpu.sync_copy(src_hbm.at[pl.ds(base, cpt)], idxs)
    pltpu.sync_copy(dst_hbm.at[pl.ds(base, cpt)], idxd)

    def start(i, buf, sem):
        pltpu.async_copy(tbl.at[idxs.at[i]], buf, sem)

    def gwait(buf, sem):
        pltpu.make_async_copy(tbl.at[idxs.at[0]], buf, sem).wait()

    def scat(i, buf):
        pltpu.sync_copy(buf, acc.at[idxd.at[i]], add=True)

    start(0, rows0, sem0)
    start(1, rows1, sem1)
    plsc.subcore_barrier()

    def body(j, _):
        i0 = 2 * j
        gwait(rows0, sem0)
        scat(i0, rows0)
        start(i0 + 2, rows0, sem0)
        gwait(rows1, sem1)
        scat(i0 + 1, rows1)
        start(i0 + 3, rows1, sem1)
        return 0

    lax.fori_loop(0, cpt // 2 - 1, body, 0)
    gwait(rows0, sem0)
    scat(cpt - 2, rows0)
    gwait(rows1, sem1)
    scat(cpt - 1, rows1)

    plsc.subcore_barrier()
    pltpu.sync_copy(
        acc.at[pl.ds(s * RPT, RPT)], out_hbm.at[c, pl.ds(s * RPT, RPT)]
    )


def _spmm_call(FH, split, xs, src2d, dst2d):
    cpt = NCHT // NS if split else NCHT // NW
    return pl.kernel(
        functools.partial(_spmm_body, FH, split),
        out_type=jax.ShapeDtypeStruct((NC, NP, FH), jnp.float32),
        mesh=_mesh(),
        scratch_types=[
            pltpu.VMEM((cpt, CH), jnp.int32),
            pltpu.VMEM((cpt, CH), jnp.int32),
            pltpu.VMEM((CH, FH), jnp.float32),
            pltpu.VMEM((CH, FH), jnp.float32),
            pltpu.VMEM_SHARED((NP, FH), jnp.float32),
            pltpu.SemaphoreType.DMA,
            pltpu.SemaphoreType.DMA,
        ],
        compiler_params=pltpu.CompilerParams(use_tc_tiling_on_sc=False),
    )(xs, src2d, dst2d)


def _dinv_body(degp_ref, dinv_ref):
    deg = 1.0 + jnp.sum(degp_ref[...], axis=0, keepdims=True)
    dinv_ref[...] = lax.rsqrt(jnp.maximum(deg, 1e-12))


def _scale_body(x_ref, d_ref, o_ref):
    xs = x_ref[...] * d_ref[...]
    o_ref[0] = xs[:, :64]
    o_ref[1] = xs[:, 64:]


def _mid_body(p0, p1, x0, x1, d, w1, b1, w2, o):
    agg0 = d[...] * (p0[...] + x0[...])
    agg1 = d[...] * (p1[...] + x1[...])
    agg = jnp.concatenate([agg0, agg1], axis=1)
    h1 = jnp.maximum(
        jnp.dot(agg, w1[...], preferred_element_type=jnp.float32) + b1[...], 0.0
    )
    g = jnp.dot(h1, w2[...], preferred_element_type=jnp.float32)
    o[...] = g * d[...]


def _head_body(q0, q1, gs, d, b2, bt, fc1w, fc1b, fc2w, fc2b, o):
    h2 = jnp.maximum(d[...] * (q0[...] + q1[...] + gs[...]) + b2[...], 0.0)
    gid = lax.broadcasted_iota(jnp.int32, (G, NP), 0)
    oh = (gid == bt[...]).astype(jnp.float32)
    psum = jnp.dot(oh, h2, preferred_element_type=jnp.float32)
    cnt = jnp.sum(oh, axis=1, keepdims=True)
    pooled = psum / jnp.maximum(cnt, 1.0)
    z = jnp.tanh(jnp.dot(pooled, fc1w[...], preferred_element_type=jnp.float32) + fc1b[...])
    zz = jnp.dot(z, fc2w[...], preferred_element_type=jnp.float32) + fc2b[...]
    o[...] = jax.nn.sigmoid(zz)


def kernel(x, edge_index, batch, W1, b1, W2, b2, fc1_w, fc1_b, fc2_w, fc2_b):
    f32 = jnp.float32
    src2d = (
        jnp.pad(edge_index[0].astype(jnp.int32), (0, EP - E), constant_values=N)
        .reshape(NCHT, CH)
    )
    dst2d = (
        jnp.pad(edge_index[1].astype(jnp.int32), (0, EP - E), constant_values=N)
        .reshape(NCHT, CH)
    )
    x_pad = jnp.pad(x.astype(f32), ((0, NP - N), (0, 0)))
    batch_pad = jnp.pad(
        batch.astype(jnp.int32), (0, NP - N), constant_values=2**20
    ).reshape(1, NP)

    degp = _deg_call(dst2d)

    dinv_row = pl.pallas_call(
        _dinv_body,
        out_shape=jax.ShapeDtypeStruct((1, NP), f32),
    )(degp)
    dinv_col = dinv_row.reshape(NP, 1)

    xs_cat = pl.pallas_call(
        _scale_body,
        out_shape=jax.ShapeDtypeStruct((NC, NP, 64), f32),
    )(x_pad, dinv_col)

    P = _spmm_call(64, True, xs_cat, src2d, dst2d)

    RB = 1280
    gs = pl.pallas_call(
        _mid_body,
        grid=(NP // RB,),
        in_specs=[
            pl.BlockSpec((RB, 64), lambda i: (i, 0)),
            pl.BlockSpec((RB, 64), lambda i: (i, 0)),
            pl.BlockSpec((RB, 64), lambda i: (i, 0)),
            pl.BlockSpec((RB, 64), lambda i: (i, 0)),
            pl.BlockSpec((RB, 1), lambda i: (i, 0)),
            pl.BlockSpec((128, 128), lambda i: (0, 0)),
            pl.BlockSpec((1, 128), lambda i: (0, 0)),
            pl.BlockSpec((128, 32), lambda i: (0, 0)),
        ],
        out_specs=pl.BlockSpec((RB, 32), lambda i: (i, 0)),
        out_shape=jax.ShapeDtypeStruct((NP, 32), f32),
    )(P[0], P[1], xs_cat[0], xs_cat[1], dinv_col, W1, b1.reshape(1, 128), W2)

    Q = _spmm_call(32, False, gs, src2d, dst2d)

    out = pl.pallas_call(
        _head_body,
        out_shape=jax.ShapeDtypeStruct((G, 1), f32),
    )(
        Q[0],
        Q[1],
        gs,
        dinv_col,
        b2.reshape(1, 32),
        batch_pad,
        fc1_w,
        fc1_b.reshape(1, 16),
        fc2_w,
        fc2_b.reshape(1, 1),
    )
    return out
